# baseline (speedup 1.0000x reference)
import functools
import math

import numpy as np
import jax
import jax.numpy as jnp
from jax import lax
from jax.experimental import pallas as pl
from jax.experimental.pallas import tpu as pltpu
from jax.experimental.pallas import tpu_sc as plsc

_NC = 2
_NS = 16
_NW = _NC * _NS


def _pe_const(seq_len: int, d_model: int) -> np.ndarray:
    position = np.arange(seq_len, dtype=np.float32)[:, None]
    div_term = np.exp(
        np.arange(0, d_model, 2, dtype=np.float32) * -(math.log(10000.0) / d_model)
    )
    pe = np.zeros((seq_len, d_model), dtype=np.float32)
    pe[:, 0::2] = np.sin(position * div_term)
    pe[:, 1::2] = np.cos(position * div_term)
    return pe


@functools.lru_cache(maxsize=None)
def _make_gather_pe_kernel(N: int, D: int, S: int, C: int):
    B = N // S
    P = S // _NW
    n_chunks = B * P // C
    cpb = P // C
    NR = 4
    mesh = plsc.VectorSubcoreMesh(core_axis_name="c", subcore_axis_name="s")

    @functools.partial(
        pl.kernel,
        out_type=jax.ShapeDtypeStruct((N, D), jnp.float32),
        mesh=mesh,
        scratch_types=[
            pltpu.VMEM((B, cpb, C), jnp.int32),
            pltpu.VMEM((NR, C, D), jnp.float32),
            pltpu.VMEM((P, D), jnp.float32),
            pltpu.SemaphoreType.DMA((NR,)),
            pltpu.SemaphoreType.DMA((NR,)),
            pltpu.SemaphoreType.DMA,
        ],
    )
    def k(table_hbm, idx_hbm, pe_hbm, out_hbm, idx_v, rows_v, pe_v, gsem, osem, psem):
        wid = lax.axis_index("s") * _NC + lax.axis_index("c")
        pe_cp = pltpu.async_copy(pe_hbm.at[pl.ds(wid * P, P)], pe_v, psem)
        for b in range(B):
            pltpu.sync_copy(idx_hbm.at[b * _NW + wid], idx_v.at[b])

        def gather(c):
            return pltpu.async_copy(
                table_hbm.at[idx_v.at[c // cpb, c % cpb]],
                rows_v.at[c % NR],
                gsem.at[c % NR],
            )

        def obase(c):
            return (c // cpb) * S + wid * P + (c % cpb) * C

        d_g = [None] * n_chunks
        d_out = [None] * n_chunks
        d_g[0] = gather(0)
        d_g[1] = gather(1)
        pe_cp.wait()
        for c in range(n_chunks):
            s = c % NR
            if c + 2 < n_chunks:
                if c >= 2:
                    d_out[c - 2].wait()
                d_g[c + 2] = gather(c + 2)
            d_g[c].wait()
            rv = rows_v.at[s]
            p0 = (c % cpb) * C

            @plsc.parallel_loop(0, C * (D // 16), unroll=4)
            def body(i):
                r = i // (D // 16)
                j = i % (D // 16)
                sl = pl.ds(j * 16, 16)
                plsc.addupdate(rv.at[r, sl], pe_v[p0 + r, sl])

            d_out[c] = pltpu.async_copy(
                rv, out_hbm.at[pl.ds(obase(c), C)], osem.at[s]
            )
        for c in range(n_chunks - 4, n_chunks):
            d_out[c].wait()

    return k


_CHUNK = 16


def kernel(table, x):
    B, S = x.shape
    V, D = table.shape
    N = B * S
    P = S // _NW
    idx = x.reshape(B * _NW, P // _CHUNK, _CHUNK).astype(jnp.int32)
    pe = jnp.asarray(_pe_const(S, D))
    out = _make_gather_pe_kernel(N, D, S, _CHUNK)(table, idx, pe)
    return out.reshape(B, S, D)


# speedup vs baseline: 1.2457x; 1.2457x over previous
import functools
import math

import numpy as np
import jax
import jax.numpy as jnp
from jax import lax
from jax.experimental import pallas as pl
from jax.experimental.pallas import tpu as pltpu
from jax.experimental.pallas import tpu_sc as plsc

_NC = 2
_NS = 16
_NW = _NC * _NS


def _pe_const(seq_len: int, d_model: int) -> np.ndarray:
    position = np.arange(seq_len, dtype=np.float32)[:, None]
    div_term = np.exp(
        np.arange(0, d_model, 2, dtype=np.float32) * -(math.log(10000.0) / d_model)
    )
    pe = np.zeros((seq_len, d_model), dtype=np.float32)
    pe[:, 0::2] = np.sin(position * div_term)
    pe[:, 1::2] = np.cos(position * div_term)
    return pe


@functools.lru_cache(maxsize=None)
def _make_gather_pe_kernel(N: int, D: int, S: int, C: int):
    b_per_w = N // _NW
    n_chunks = b_per_w // C
    mesh = plsc.VectorSubcoreMesh(core_axis_name="c", subcore_axis_name="s")

    @functools.partial(
        pl.kernel,
        out_type=jax.ShapeDtypeStruct((N, D), jnp.float32),
        mesh=mesh,
        scratch_types=[
            pltpu.VMEM((n_chunks, C), jnp.int32),
            pltpu.VMEM((2, C, D), jnp.float32),
            pltpu.VMEM((2, C, D), jnp.float32),
            pltpu.SemaphoreType.DMA((2,)),
            pltpu.SemaphoreType.DMA((2,)),
            pltpu.SemaphoreType.DMA((2,)),
        ],
    )
    def k(table_hbm, idx_hbm, pe_hbm, out_hbm, idx_v, rows_v, pe_v, gsem, psem, osem):
        wid = lax.axis_index("s") * _NC + lax.axis_index("c")
        base = wid * b_per_w
        pos0 = base % S
        pltpu.sync_copy(idx_hbm.at[wid], idx_v)

        def issue(c):
            s = c % 2
            pc = pltpu.async_copy(pe_hbm.at[pl.ds(pos0 + c * C, C)], pe_v.at[s], psem.at[s])
            gc = pltpu.async_copy(table_hbm.at[idx_v.at[c]], rows_v.at[s], gsem.at[s])
            return pc, gc

        d_in = [None] * n_chunks
        d_out = [None] * n_chunks
        d_in[0] = issue(0)
        for c in range(n_chunks):
            s = c % 2
            if c + 1 < n_chunks:
                if c >= 1:
                    d_out[c - 1].wait()
                d_in[c + 1] = issue(c + 1)
            pc, gc = d_in[c]
            pc.wait()
            gc.wait()
            rv = rows_v.at[s]
            pv = pe_v.at[s]

            def body(r, _):
                for j in range(D // 16):
                    sl = pl.ds(j * 16, 16)
                    rv[r, sl] = rv[r, sl] + pv[r, sl]
                return 0

            lax.fori_loop(0, C, body, 0)
            d_out[c] = pltpu.async_copy(rv, out_hbm.at[pl.ds(base + c * C, C)], osem.at[s])
        d_out[n_chunks - 2].wait()
        d_out[n_chunks - 1].wait()

    return k


_CHUNK = 32


def kernel(table, x):
    B, S = x.shape
    V, D = table.shape
    N = B * S
    idx = x.reshape(_NW, N // _NW // _CHUNK, _CHUNK).astype(jnp.int32)
    pe = jnp.asarray(_pe_const(S, D))
    out = _make_gather_pe_kernel(N, D, S, _CHUNK)(table, idx, pe)
    return out.reshape(B, S, D)
